# baseline (speedup 1.0000x reference)
import functools

import jax
import jax.numpy as jnp
from jax import lax
from jax.experimental import pallas as pl
from jax.experimental.pallas import tpu as pltpu
from jax.experimental.pallas import tpu_sc as plsc

NUM_CORES = 2
NUM_SUBCORES = 16
LANES = 16
NUM_WORKERS = NUM_CORES * NUM_SUBCORES

CHUNK = 16


def _body(x_hbm, spans_hbm, pe_hbm, out_hbm, idx_v, x_buf, pe_buf,
          gsem, xsem):
    rows_per_w = x_hbm.shape[0] // NUM_WORKERS
    n_chunks = rows_per_w // CHUNK
    hidden = x_hbm.shape[1]
    segs = hidden // LANES

    wid = lax.axis_index("s") * NUM_CORES + lax.axis_index("c")
    base = wid * rows_per_w

    pltpu.sync_copy(spans_hbm.at[pl.ds(base, rows_per_w)], idx_v)

    def chunk_body(ci, carry):
        row0 = base + ci * CHUNK
        ioff = pl.multiple_of(ci * CHUNK, CHUNK)
        gather = pltpu.async_copy(
            pe_hbm.at[idx_v.at[pl.ds(ioff, CHUNK)]], pe_buf, gsem)
        xload = pltpu.async_copy(x_hbm.at[pl.ds(row0, CHUNK)], x_buf, xsem)
        gather.wait()
        xload.wait()

        def row_body(r, c):
            def seg_body(j, c2):
                seg = pl.ds(pl.multiple_of(j * LANES, LANES), LANES)
                plsc.addupdate(x_buf.at[r, seg], pe_buf[r, seg])
                return c2
            return lax.fori_loop(0, segs, seg_body, c)

        lax.fori_loop(0, CHUNK, row_body, carry)
        pltpu.sync_copy(x_buf, out_hbm.at[pl.ds(row0, CHUNK)])
        return carry

    lax.fori_loop(0, n_chunks, chunk_body, 0)


@jax.jit
def _pe_add(x2, spans1, pe2):
    n_rows, hidden = x2.shape
    mesh = plsc.VectorSubcoreMesh(
        core_axis_name="c", subcore_axis_name="s",
        num_cores=NUM_CORES, num_subcores=NUM_SUBCORES)
    rows_per_w = n_rows // NUM_WORKERS
    return pl.kernel(
        _body,
        out_type=jax.ShapeDtypeStruct((n_rows, hidden), jnp.float32),
        mesh=mesh,
        scratch_types=[
            pltpu.VMEM((rows_per_w,), jnp.int32),
            pltpu.VMEM((CHUNK, hidden), jnp.float32),
            pltpu.VMEM((CHUNK, hidden), jnp.float32),
            pltpu.SemaphoreType.DMA,
            pltpu.SemaphoreType.DMA,
        ],
    )(x2, spans1, pe2)


def kernel(x, spans, pe):
    b, l, h = x.shape
    x2 = x.reshape(b * l, h)
    spans1 = spans.reshape(b * l).astype(jnp.int32)
    pe2 = pe[0]
    out = _pe_add(x2, spans1, pe2)
    return out.reshape(b, l, h)


# speedup vs baseline: 2.5954x; 2.5954x over previous
import jax
import jax.numpy as jnp
from jax import lax
from jax.experimental import pallas as pl
from jax.experimental.pallas import tpu as pltpu
from jax.experimental.pallas import tpu_sc as plsc

NUM_CORES = 2
NUM_SUBCORES = 16
LANES = 16
NUM_WORKERS = NUM_CORES * NUM_SUBCORES

CHUNK = 8
NBUF = 3


def _body(x_hbm, spans_hbm, pe_hbm, out_hbm,
          idx_v, xb0, xb1, xb2, pb0, pb1, pb2,
          gs0, gs1, gs2, xs0, xs1, xs2, os0, os1, os2):
    x_bufs = (xb0, xb1, xb2)
    pe_bufs = (pb0, pb1, pb2)
    gsems = (gs0, gs1, gs2)
    xsems = (xs0, xs1, xs2)
    osems = (os0, os1, os2)

    rows_per_w = x_hbm.shape[0] // NUM_WORKERS
    n_chunks = rows_per_w // CHUNK
    hidden = x_hbm.shape[1]

    wid = lax.axis_index("s") * NUM_CORES + lax.axis_index("c")
    base = wid * rows_per_w

    def idx_slice(ci):
        return idx_v.at[pl.ds(pl.multiple_of(ci * CHUNK, CHUNK), CHUNK)]

    def row_slice(hbm, ci):
        return hbm.at[pl.ds(base + ci * CHUNK, CHUNK)]

    def drain(dst, sem):
        pltpu.make_async_copy(x_hbm.at[pl.ds(0, CHUNK)], dst, sem).wait()

    pltpu.sync_copy(spans_hbm.at[pl.ds(base, rows_per_w)], idx_v)

    for s in range(NBUF):
        pltpu.async_copy(pe_hbm.at[idx_slice(s)], pe_bufs[s], gsems[s])
    for s in range(NBUF - 1):
        pltpu.async_copy(row_slice(x_hbm, s), x_bufs[s], xsems[s])

    n_groups = (n_chunks + NBUF) // NBUF

    def group_body(g, carry):
        for b in range(NBUF):
            ci = g * NBUF + b

            @pl.when(ci < n_chunks)
            def _():
                drain(pe_bufs[b], gsems[b])
                drain(x_bufs[b], xsems[b])
                x_b, pe_b = x_bufs[b], pe_bufs[b]
                for r in range(CHUNK):
                    @plsc.parallel_loop(0, hidden, step=LANES, unroll=8)
                    def _(i):
                        seg = pl.ds(pl.multiple_of(i, LANES), LANES)
                        plsc.addupdate(x_b.at[r, seg], pe_b[r, seg])
                pltpu.async_copy(x_b, row_slice(out_hbm, ci), osems[b])
                @pl.when(ci + NBUF < n_chunks)
                def _():
                    pltpu.async_copy(
                        pe_hbm.at[idx_slice(ci + NBUF)], pe_b, gsems[b])

            d = (b + 2) % NBUF

            @pl.when(jnp.logical_and(ci >= 1, ci - 1 < n_chunks))
            def _():
                drain(x_bufs[d], osems[d])

            @pl.when(ci + 2 < n_chunks)
            def _():
                pltpu.async_copy(
                    row_slice(x_hbm, ci + 2), x_bufs[d], xsems[d])
        return carry

    lax.fori_loop(0, n_groups, group_body, 0)


@jax.jit
def _pe_add(x2, spans1, pe2):
    n_rows, hidden = x2.shape
    mesh = plsc.VectorSubcoreMesh(
        core_axis_name="c", subcore_axis_name="s",
        num_cores=NUM_CORES, num_subcores=NUM_SUBCORES)
    rows_per_w = n_rows // NUM_WORKERS
    return pl.kernel(
        _body,
        out_type=jax.ShapeDtypeStruct((n_rows, hidden), jnp.float32),
        mesh=mesh,
        scratch_types=(
            [pltpu.VMEM((rows_per_w,), jnp.int32)]
            + [pltpu.VMEM((CHUNK, hidden), jnp.float32)] * (2 * NBUF)
            + [pltpu.SemaphoreType.DMA] * (3 * NBUF)
        ),
    )(x2, spans1, pe2)


def kernel(x, spans, pe):
    b, l, h = x.shape
    x2 = x.reshape(b * l, h)
    spans1 = spans.reshape(b * l).astype(jnp.int32)
    pe2 = pe[0]
    out = _pe_add(x2, spans1, pe2)
    return out.reshape(b, l, h)


# speedup vs baseline: 2.6241x; 1.0110x over previous
import jax
import jax.numpy as jnp
from jax import lax
from jax.experimental import pallas as pl
from jax.experimental.pallas import tpu as pltpu
from jax.experimental.pallas import tpu_sc as plsc

NUM_CORES = 2
NUM_SUBCORES = 16
LANES = 16
NUM_WORKERS = NUM_CORES * NUM_SUBCORES

CHUNK = 8
NBUF = 3


def _body(x_hbm, spans_hbm, pe_hbm, out_hbm,
          idx_v, xb0, xb1, xb2, pb0, pb1, pb2,
          gs0, gs1, gs2, xs0, xs1, xs2, os0, os1, os2):
    x_bufs = (xb0, xb1, xb2)
    pe_bufs = (pb0, pb1, pb2)
    gsems = (gs0, gs1, gs2)
    xsems = (xs0, xs1, xs2)
    osems = (os0, os1, os2)

    batch, seq, hidden = x_hbm.shape
    rows_per_w = (batch * seq) // NUM_WORKERS
    n_chunks = rows_per_w // CHUNK
    w_per_batch = seq // rows_per_w

    wid = lax.axis_index("s") * NUM_CORES + lax.axis_index("c")
    b_ix = wid // w_per_batch
    base = (wid % w_per_batch) * rows_per_w
    pe2 = pe_hbm.at[0]

    def idx_slice(ci):
        return idx_v.at[pl.ds(pl.multiple_of(ci * CHUNK, CHUNK), CHUNK)]

    def row_slice(hbm, ci):
        return hbm.at[b_ix, pl.ds(base + ci * CHUNK, CHUNK)]

    def drain(dst, sem):
        pltpu.make_async_copy(x_hbm.at[0, pl.ds(0, CHUNK)], dst, sem).wait()

    pltpu.sync_copy(spans_hbm.at[b_ix, pl.ds(base, rows_per_w)], idx_v)

    for s in range(NBUF):
        pltpu.async_copy(pe2.at[idx_slice(s)], pe_bufs[s], gsems[s])
    for s in range(NBUF - 1):
        pltpu.async_copy(row_slice(x_hbm, s), x_bufs[s], xsems[s])

    n_groups = (n_chunks + NBUF) // NBUF

    def group_body(g, carry):
        for b in range(NBUF):
            ci = g * NBUF + b

            @pl.when(ci < n_chunks)
            def _():
                drain(pe_bufs[b], gsems[b])
                drain(x_bufs[b], xsems[b])
                x_b, pe_b = x_bufs[b], pe_bufs[b]
                for r in range(CHUNK):
                    @plsc.parallel_loop(0, hidden, step=LANES, unroll=16)
                    def _(i):
                        seg = pl.ds(pl.multiple_of(i, LANES), LANES)
                        plsc.addupdate(x_b.at[r, seg], pe_b[r, seg])
                pltpu.async_copy(x_b, row_slice(out_hbm, ci), osems[b])
                @pl.when(ci + NBUF < n_chunks)
                def _():
                    pltpu.async_copy(
                        pe2.at[idx_slice(ci + NBUF)], pe_b, gsems[b])

            d = (b + 2) % NBUF

            @pl.when(jnp.logical_and(ci >= 1, ci - 1 < n_chunks))
            def _():
                drain(x_bufs[d], osems[d])

            @pl.when(ci + 2 < n_chunks)
            def _():
                pltpu.async_copy(
                    row_slice(x_hbm, ci + 2), x_bufs[d], xsems[d])
        return carry

    lax.fori_loop(0, n_groups, group_body, 0)


def kernel(x, spans, pe):
    batch, seq, hidden = x.shape
    rows_per_w = (batch * seq) // NUM_WORKERS
    mesh = plsc.VectorSubcoreMesh(
        core_axis_name="c", subcore_axis_name="s",
        num_cores=NUM_CORES, num_subcores=NUM_SUBCORES)
    return pl.kernel(
        _body,
        out_type=jax.ShapeDtypeStruct((batch, seq, hidden), jnp.float32),
        mesh=mesh,
        scratch_types=(
            [pltpu.VMEM((rows_per_w,), jnp.int32)]
            + [pltpu.VMEM((CHUNK, hidden), jnp.float32)] * (2 * NBUF)
            + [pltpu.SemaphoreType.DMA] * (3 * NBUF)
        ),
    )(x, spans.astype(jnp.int32), pe)


# speedup vs baseline: 2.6267x; 1.0010x over previous
import jax
import jax.numpy as jnp
from jax import lax
from jax.experimental import pallas as pl
from jax.experimental.pallas import tpu as pltpu
from jax.experimental.pallas import tpu_sc as plsc

NUM_CORES = 2
NUM_SUBCORES = 16
LANES = 16
NUM_WORKERS = NUM_CORES * NUM_SUBCORES

CHUNK = 8
NBUF = 3


def _body(x_hbm, spans_hbm, pe_hbm, out_hbm, idx_v, *scratch):
    x_bufs = scratch[:NBUF]
    pe_bufs = scratch[NBUF:2 * NBUF]
    gsems = scratch[2 * NBUF:3 * NBUF]
    xsems = scratch[3 * NBUF:4 * NBUF]
    osems = scratch[4 * NBUF:5 * NBUF]

    batch, seq, hidden = x_hbm.shape
    rows_per_w = (batch * seq) // NUM_WORKERS
    n_chunks = rows_per_w // CHUNK
    w_per_batch = seq // rows_per_w

    wid = lax.axis_index("s") * NUM_CORES + lax.axis_index("c")
    b_ix = wid // w_per_batch
    base = (wid % w_per_batch) * rows_per_w
    pe2 = pe_hbm.at[0]

    def idx_slice(ci):
        return idx_v.at[pl.ds(pl.multiple_of(ci * CHUNK, CHUNK), CHUNK)]

    def row_slice(hbm, ci):
        return hbm.at[b_ix, pl.ds(base + ci * CHUNK, CHUNK)]

    def drain(dst, sem):
        pltpu.make_async_copy(x_hbm.at[0, pl.ds(0, CHUNK)], dst, sem).wait()

    pltpu.sync_copy(spans_hbm.at[b_ix, pl.ds(base, rows_per_w)], idx_v)

    for s in range(NBUF):
        pltpu.async_copy(pe2.at[idx_slice(s)], pe_bufs[s], gsems[s])
    for s in range(NBUF - 1):
        pltpu.async_copy(row_slice(x_hbm, s), x_bufs[s], xsems[s])

    n_groups = (n_chunks + NBUF) // NBUF

    def group_body(g, carry):
        for b in range(NBUF):
            ci = g * NBUF + b

            @pl.when(ci < n_chunks)
            def _():
                drain(pe_bufs[b], gsems[b])
                drain(x_bufs[b], xsems[b])
                x_b, pe_b = x_bufs[b], pe_bufs[b]
                for r in range(CHUNK):
                    @plsc.parallel_loop(0, hidden, step=LANES, unroll=16)
                    def _(i):
                        seg = pl.ds(pl.multiple_of(i, LANES), LANES)
                        plsc.addupdate(x_b.at[r, seg], pe_b[r, seg])
                pltpu.async_copy(x_b, row_slice(out_hbm, ci), osems[b])
                @pl.when(ci + NBUF < n_chunks)
                def _():
                    pltpu.async_copy(
                        pe2.at[idx_slice(ci + NBUF)], pe_b, gsems[b])

            d = (b + 2) % NBUF

            @pl.when(jnp.logical_and(ci >= 1, ci - 1 < n_chunks))
            def _():
                drain(x_bufs[d], osems[d])

            @pl.when(ci + 2 < n_chunks)
            def _():
                pltpu.async_copy(
                    row_slice(x_hbm, ci + 2), x_bufs[d], xsems[d])
        return carry

    lax.fori_loop(0, n_groups, group_body, 0)


def kernel(x, spans, pe):
    batch, seq, hidden = x.shape
    rows_per_w = (batch * seq) // NUM_WORKERS
    mesh = plsc.VectorSubcoreMesh(
        core_axis_name="c", subcore_axis_name="s",
        num_cores=NUM_CORES, num_subcores=NUM_SUBCORES)
    return pl.kernel(
        _body,
        out_type=jax.ShapeDtypeStruct((batch, seq, hidden), jnp.float32),
        mesh=mesh,
        scratch_types=(
            [pltpu.VMEM((rows_per_w,), jnp.int32)]
            + [pltpu.VMEM((CHUNK, hidden), jnp.float32)] * (2 * NBUF)
            + [pltpu.SemaphoreType.DMA] * (3 * NBUF)
        ),
    )(x, spans.astype(jnp.int32), pe)
